# baseline (speedup 1.0000x reference)
import jax
import jax.numpy as jnp
from jax import lax
from jax.experimental import pallas as pl
from jax.experimental.pallas import tpu as pltpu

_VMEM_LIMIT = 56 * 1024 * 1024
_BF = jnp.bfloat16
_F32 = jnp.float32


def _nn_up(x, s):
    if s == 1:
        return x
    h, w, c = x.shape
    x = jnp.broadcast_to(x[:, :, None, :], (h, w, s, c)).reshape(h, w * s, c)
    x = jnp.broadcast_to(x[:, None, :, :], (h, s, w * s, c)).reshape(h * s, w * s, c)
    return x


def _latdot(x, w):
    return lax.dot_general(x.astype(_BF), w, (((0,), (0,)), ((), ())),
                           preferred_element_type=_F32)


def _lat_top_kernel(x_ref, w_ref, o_ref):
    o_ref[0] = _latdot(x_ref[0], w_ref[...]).astype(o_ref.dtype)


def _lat_top(x_flat, wt, tt):
    N, Cin, HW = x_flat.shape
    Cout = wt.shape[1]
    return pl.pallas_call(
        _lat_top_kernel,
        out_shape=jax.ShapeDtypeStruct((N, HW, Cout), _BF),
        grid=(N, HW // tt),
        in_specs=[pl.BlockSpec((1, Cin, tt), lambda n, t: (n, 0, t)),
                  pl.BlockSpec((Cin, Cout), lambda n, t: (0, 0))],
        out_specs=pl.BlockSpec((1, tt, Cout), lambda n, t: (n, t, 0)),
        compiler_params=pltpu.CompilerParams(
            dimension_semantics=("parallel", "parallel"),
            vmem_limit_bytes=_VMEM_LIMIT),
        cost_estimate=pl.CostEstimate(
            flops=2 * N * HW * Cin * Cout, transcendentals=0,
            bytes_accessed=4 * N * Cin * HW + 2 * N * HW * Cout),
    )(x_flat, wt)


def _lat_chain_kernel(c4_ref, c3_ref, c2_ref, t5_ref, w_ref,
                      o4_ref, o3_ref, o2_ref, ho_ref):
    hb = pl.program_id(1)
    th4, W4 = o4_ref.shape[1], o4_ref.shape[2]
    W3, W2 = o3_ref.shape[2], o2_ref.shape[2]
    t5 = t5_ref[0, pl.ds((th4 // 2) * hb, th4 // 2)].astype(_F32)
    v4 = _latdot(c4_ref[0], w_ref[0:1024]).reshape(o4_ref.shape[1:]) + _nn_up(t5, 2)
    o4_ref[0] = v4.astype(o4_ref.dtype)
    ho_ref[0, 0, 0, 0:W4] = v4[0].astype(ho_ref.dtype)
    ho_ref[0, 0, 1, 0:W4] = v4[-1].astype(ho_ref.dtype)
    v3 = _latdot(c3_ref[0], w_ref[1024:1536]).reshape(o3_ref.shape[1:]) + _nn_up(v4, 2)
    o3_ref[0] = v3.astype(o3_ref.dtype)
    ho_ref[0, 0, 0, W4:W4 + W3] = v3[0].astype(ho_ref.dtype)
    ho_ref[0, 0, 1, W4:W4 + W3] = v3[-1].astype(ho_ref.dtype)
    th2 = o2_ref.shape[1]
    up3 = _nn_up(v3, 2)
    w2 = w_ref[1536:1792]
    first = last = None
    for h in range(th2):
        row = _latdot(c2_ref[0, :, h, :], w2) + up3[h]
        o2_ref[0, h] = row.astype(o2_ref.dtype)
        if h == 0:
            first = row
        if h == th2 - 1:
            last = row
    ho_ref[0, 0, 0, W4 + W3:W4 + W3 + W2] = first.astype(ho_ref.dtype)
    ho_ref[0, 0, 1, W4 + W3:W4 + W3 + W2] = last.astype(ho_ref.dtype)


def _lat_chain(c4f, c3f, c2_4d, t5, wcat, *, th2=16):
    N = c2_4d.shape[0]
    W2 = t5.shape[2] * 8
    H2 = c2_4d.shape[2]
    H5, W5 = t5.shape[1], t5.shape[2]
    H4, W4, H3, W3 = H2 // 4, W2 // 4, H2 // 2, W2 // 2
    th4, th3 = th2 // 4, th2 // 2
    nblk = H2 // th2
    Wcat = W4 + W3 + W2
    flops = 2 * N * 256 * (H2 * W2 * 256 + H3 * W3 * 512 + H4 * W4 * 1024)
    out_shapes = [jax.ShapeDtypeStruct((N, H4, W4, 256), _BF),
                  jax.ShapeDtypeStruct((N, H3, W3, 256), _BF),
                  jax.ShapeDtypeStruct((N, H2, W2, 256), _BF),
                  jax.ShapeDtypeStruct((N, nblk, 2, Wcat, 256), _BF)]
    return pl.pallas_call(
        _lat_chain_kernel,
        out_shape=out_shapes,
        grid=(N, nblk),
        in_specs=[
            pl.BlockSpec((1, 1024, th4 * W4), lambda n, h: (n, 0, h)),
            pl.BlockSpec((1, 512, th3 * W3), lambda n, h: (n, 0, h)),
            pl.BlockSpec((1, 256, th2, W2), lambda n, h: (n, 0, h, 0)),
            pl.BlockSpec((1, H5, W5, 256), lambda n, h: (n, 0, 0, 0)),
            pl.BlockSpec((1792, 256), lambda n, h: (0, 0)),
        ],
        out_specs=[
            pl.BlockSpec((1, th4, W4, 256), lambda n, h: (n, h, 0, 0)),
            pl.BlockSpec((1, th3, W3, 256), lambda n, h: (n, h, 0, 0)),
            pl.BlockSpec((1, th2, W2, 256), lambda n, h: (n, h, 0, 0)),
            pl.BlockSpec((1, 1, 2, Wcat, 256), lambda n, h: (n, h, 0, 0, 0)),
        ],
        compiler_params=pltpu.CompilerParams(
            dimension_semantics=("parallel", "parallel"),
            vmem_limit_bytes=_VMEM_LIMIT),
        cost_estimate=pl.CostEstimate(
            flops=flops, transcendentals=0,
            bytes_accessed=4 * (N * 1024 * H4 * W4 + N * 512 * H3 * W3
                                + N * 256 * H2 * W2)
            + 2 * N * 256 * (H4 * W4 + H3 * W3 + H2 * W2)),
    )(c4f, c3f, c2_4d, t5, wcat)


_SCALES = (8, 4, 2, 1)
_TH2 = 16


def _fuse_kernel(t5_ref, c4c, c3c, c2c, hp, hn, w_ref, o_ref, s5, s4, s3, s2):
    hb = pl.program_id(1)
    nblk = pl.num_programs(1)
    H5 = t5_ref.shape[1]
    W5 = t5_ref.shape[2]
    W4, W3, W2 = 2 * W5, 4 * W5, 8 * W5
    stks = (s5, s4, s3, s2)
    for idx in range(4):
        scale = _SCALES[idx]
        th = _TH2 // scale
        Br = stks[idx]
        if idx == 0:
            W = W5
            xc = t5_ref[0, pl.ds(th * hb, th)]
            top = t5_ref[0, pl.ds(jnp.maximum(th * hb - 1, 0), 1)][0]
            bot = t5_ref[0, pl.ds(jnp.minimum(th * hb + th, H5 - 1), 1)][0]
        else:
            xc = (c4c, c3c, c2c)[idx - 1][0]
            W = xc.shape[1]
            off = {W4: 0, W3: W4, W2: W4 + W3}[W]
            top = hp[0, 0, 1, off:off + W]
            bot = hn[0, 0, 0, off:off + W]
        T = th * W
        z = jnp.zeros((W, 256), _BF)
        Br[:, 256:512] = xc.reshape(T, 256)
        Br[W:, 0:256] = xc[0:th - 1].reshape(T - W, 256)
        Br[:T - W, 512:768] = xc[1:th].reshape(T - W, 256)
        Br[0:W, 0:256] = jnp.where(hb == 0, z, top.reshape(W, 256))
        Br[T - W:, 512:768] = jnp.where(hb == nblk - 1, z, bot.reshape(W, 256))
        S = jnp.dot(Br[...], w_ref[idx],
                    preferred_element_type=_F32).reshape(th, W, 384)
        zc = jnp.zeros((th, 1, 128), _F32)
        y = (S[:, :, 128:256]
             + jnp.concatenate([zc, S[:, :-1, 0:128]], axis=1)
             + jnp.concatenate([S[:, 1:, 256:384], zc], axis=1))
        up = _nn_up(y, scale)
        hw = up.shape[0] * up.shape[1]
        yt = jnp.transpose(up.reshape(hw, 128))[0:64]
        o_ref[0, 64 * idx:64 * (idx + 1), :] = yt.astype(o_ref.dtype)


def _fused_convs_concat(in5, out4, out3, out2, ho, wstk):
    N, H2, W2, Cb = out2.shape[0], out2.shape[1], out2.shape[2], 64
    H5, W5 = in5.shape[1], in5.shape[2]
    nblk = H2 // _TH2
    Wcat = ho.shape[3]
    flops = sum(2 * 9 * N * (H2 // s) * (W2 // s) * 256 * 64 for s in _SCALES)
    stks = [pltpu.VMEM(((_TH2 // s) * (W2 // s), 768), _BF) for s in _SCALES]
    specs = [
        pl.BlockSpec((1, H5, W5, 256), lambda n, h: (n, 0, 0, 0)),
        pl.BlockSpec((1, _TH2 // 4, W2 // 4, 256), lambda n, h: (n, h, 0, 0)),
        pl.BlockSpec((1, _TH2 // 2, W2 // 2, 256), lambda n, h: (n, h, 0, 0)),
        pl.BlockSpec((1, _TH2, W2, 256), lambda n, h: (n, h, 0, 0)),
        pl.BlockSpec((1, 1, 2, Wcat, 256),
                     lambda n, h: (n, jnp.maximum(h - 1, 0), 0, 0, 0)),
        pl.BlockSpec((1, 1, 2, Wcat, 256),
                     lambda n, h, nb=nblk: (n, jnp.minimum(h + 1, nb - 1), 0, 0, 0)),
        pl.BlockSpec((4, 768, 384), lambda n, h: (0, 0, 0)),
    ]
    return pl.pallas_call(
        _fuse_kernel,
        out_shape=jax.ShapeDtypeStruct((N, 4 * Cb, H2 * W2), _F32),
        grid=(N, nblk),
        in_specs=specs,
        out_specs=pl.BlockSpec((1, 4 * Cb, _TH2 * W2), lambda n, h: (n, 0, h)),
        scratch_shapes=stks,
        compiler_params=pltpu.CompilerParams(
            dimension_semantics=("parallel", "parallel"),
            vmem_limit_bytes=_VMEM_LIMIT),
        cost_estimate=pl.CostEstimate(
            flops=flops, transcendentals=0,
            bytes_accessed=4 * N * H2 * W2 * 4 * Cb
            + 2 * N * (H2 * W2 + (H2 // 2) * (W2 // 2)) * 256),
    )(in5, out4, out3, out2, ho, ho, wstk)


def _mk_conv_w(p):
    wt = jnp.transpose(p, (2, 3, 1, 0))
    wt = jnp.pad(wt, ((0, 0), (0, 0), (0, 0), (0, 64)))
    return jnp.transpose(wt, (0, 2, 1, 3)).reshape(768, 384).astype(_BF)


def kernel(c2, c3, c4, c5, in2, in3, in4, in5, p5, p4, p3, p2):
    N = c2.shape[0]
    w5t = jnp.transpose(in5).astype(_BF)
    wcat = jnp.concatenate([jnp.transpose(in4), jnp.transpose(in3),
                            jnp.transpose(in2)], axis=0).astype(_BF)
    wstk = jnp.stack([_mk_conv_w(p5), _mk_conv_w(p4),
                      _mk_conv_w(p3), _mk_conv_w(p2)])

    c5f = c5.reshape(N, c5.shape[1], -1)
    c4f = c4.reshape(N, c4.shape[1], -1)
    c3f = c3.reshape(N, c3.shape[1], -1)
    c2f = c2.reshape(N, c2.shape[1], -1)

    t5 = _lat_top(c5f, w5t, min(512, c5f.shape[2]))
    t5 = t5.reshape(N, c5.shape[2], c5.shape[3], 256)
    o4, o3, o2, ho = _lat_chain(c4f, c3f, c2, t5, wcat,
                                th2=min(16, c2.shape[2]))

    fuse = _fused_convs_concat(t5, o4, o3, o2, ho, wstk)
    return fuse.reshape(N, 256, c2.shape[2], c2.shape[3])
